# baseline (speedup 1.0000x reference)
import functools

import jax
import jax.numpy as jnp
from jax import lax
from jax.experimental import pallas as pl
from jax.experimental.pallas import tpu as pltpu
from jax.experimental.pallas import tpu_sc as plsc

N = 50000
NP = 50048
D = 64
C = 32
E = 800000
NC, NS, L = 2, 16, 16

EPAD = 5568 * 144 - E
EC = 144
EROWS2 = 5568
UPT = EROWS2 // NS
SLOTS = 6
ROUNDS = UPT // SLOTS
RPT_N = NP // NS
RB = 184
NRB = RPT_N // RB
ALPHA = 1.0 / 4.0


def _bcast(w16, e):
    return jnp.take_along_axis(w16, jnp.full((16,), e, jnp.int32), axis=0)


def _body(ux0, rx0, ep_ur, ep_ru,
          uout, rout, xu, xr,
          acc, rows, epk, gsem, ssem, isem):
    cid = lax.axis_index("c")
    sid = lax.axis_index("s")
    rbase = sid * RPT_N
    TA, TB = 0, RB

    ux0c, rx0c = ux0.at[cid], rx0.at[cid]
    uoutc, routc = uout.at[cid], rout.at[cid]
    xuc, xrc = xu.at[cid], xr.at[cid]

    def initblk(b, carry):
        off = pl.multiple_of(rbase + b * RB, 8)
        sl = pl.ds(off, RB)
        pltpu.sync_copy(ux0c.at[sl], rows.at[pl.ds(TA, RB)])
        pltpu.sync_copy(rx0c.at[sl], rows.at[pl.ds(TB, RB)])
        pltpu.sync_copy(rows.at[pl.ds(TA, RB)], xuc.at[sl])

        def irow(r, c2):
            rows[TA + r, 0:16] = rows[TA + r, 0:16] * ALPHA
            rows[TA + r, 16:32] = rows[TA + r, 16:32] * ALPHA
            rows[TB + r, 0:16] = rows[TB + r, 0:16] * ALPHA
            rows[TB + r, 16:32] = rows[TB + r, 16:32] * ALPHA
            return c2
        lax.fori_loop(0, RB, irow, 0, unroll=2)
        pltpu.sync_copy(rows.at[pl.ds(TA, RB)], uoutc.at[sl])
        pltpu.sync_copy(rows.at[pl.ds(TB, RB)], routc.at[sl])
        return carry
    lax.fori_loop(0, NRB, initblk, 0)

    def zrow(r, carry):
        z16 = lax.broadcast(jnp.float32(0.0), (16,))
        rows[r, 0:16] = z16
        rows[r, 16:32] = z16
        return carry
    lax.fori_loop(0, RB, zrow, 0, unroll=8)

    def zacc(b, carry):
        off = pl.multiple_of(rbase + b * RB, 8)
        pltpu.sync_copy(rows.at[pl.ds(TA, RB)], acc.at[pl.ds(off, RB)])
        return carry
    lax.fori_loop(0, NRB, zacc, 0)
    plsc.subcore_barrier()

    ubase = sid * UPT

    def scatter_phase(xsrc, ep):
        def load_idx(u, s):
            pltpu.async_copy(ep.at[pl.ds(u, 1)], epk.at[pl.ds(s, 1)],
                             isem.at[s])

        def wait_idx(s):
            pltpu.make_async_copy(ep.at[pl.ds(ubase, 1)],
                                  epk.at[pl.ds(s, 1)], isem.at[s]).wait()

        def gather(s):
            pltpu.async_copy(xsrc.at[epk.at[s, 0]],
                             rows.at[pl.ds(s * EC, EC)], gsem.at[s])

        def wait_gather(s):
            pltpu.make_async_copy(xsrc.at[epk.at[s, 0]],
                                  rows.at[pl.ds(s * EC, EC)],
                                  gsem.at[s]).wait()

        def scatter(s):
            pltpu.async_copy(rows.at[pl.ds(s * EC, EC)],
                             acc.at[epk.at[s, 1]], ssem.at[s], add=True)

        def wait_scatter(s):
            pltpu.make_async_copy(rows.at[pl.ds(s * EC, EC)],
                                  acc.at[epk.at[s, 1]], ssem.at[s]).wait()

        def mul_unit(s):
            def mgrp(g, carry2):
                w16 = lax.bitcast_convert_type(epk[s, 2, pl.ds(g * 16, 16)],
                                               jnp.float32)
                base = s * EC + g * 16
                for e in range(16):
                    wb = _bcast(w16, e)
                    rows[base + e, 0:16] = rows[base + e, 0:16] * wb
                    rows[base + e, 16:32] = rows[base + e, 16:32] * wb
                return carry2
            lax.fori_loop(0, EC // 16, mgrp, 0, unroll=3)

        def zslot(r, carry):
            z16 = lax.broadcast(jnp.float32(0.0), (16,))
            rows[4 * EC + r, 0:16] = z16
            rows[4 * EC + r, 16:32] = z16
            return carry
        lax.fori_loop(0, 2 * EC, zslot, 0, unroll=4)
        for s in range(3):
            pltpu.sync_copy(ep.at[pl.ds(ubase + s, 1)], epk.at[pl.ds(s, 1)])
            gather(s)
        load_idx(ubase + 3, 3)
        for s in (4, 5):
            pltpu.sync_copy(ep.at[pl.ds(ubase, 1)], epk.at[pl.ds(s, 1)])
            scatter(s)

        def round_body(r, carry):
            h0 = r * SLOTS
            for k in range(SLOTS):
                k3 = (k + 3) % SLOTS
                k4 = (k + 4) % SLOTS
                wait_gather(k)
                mul_unit(k)
                scatter(k)
                wait_idx(k3)
                gather(k3)
                wait_scatter(k4)
                load_idx(ubase + lax.rem(h0 + k + 4, UPT), k4)
            return carry
        lax.fori_loop(0, ROUNDS, round_body, 0)

        for s in range(3):
            wait_gather(s)
        wait_idx(3)
        wait_scatter(4)
        wait_scatter(5)

    def drain(out_ref, xdst):
        def dblk(b, carry):
            off = pl.multiple_of(rbase + b * RB, 8)
            sl = pl.ds(off, RB)
            pltpu.sync_copy(acc.at[sl], rows.at[pl.ds(TA, RB)])
            pltpu.sync_copy(rows.at[pl.ds(TA, RB)], xdst.at[sl])
            pltpu.sync_copy(out_ref.at[sl], rows.at[pl.ds(TB, RB)])

            def urow(r, carry2):
                z16 = lax.broadcast(jnp.float32(0.0), (16,))
                a0 = rows[TA + r, 0:16]
                a1 = rows[TA + r, 16:32]
                rows[TB + r, 0:16] = rows[TB + r, 0:16] + a0 * ALPHA
                rows[TB + r, 16:32] = rows[TB + r, 16:32] + a1 * ALPHA
                rows[TA + r, 0:16] = z16
                rows[TA + r, 16:32] = z16
                return carry2
            lax.fori_loop(0, RB, urow, 0, unroll=2)
            pltpu.sync_copy(rows.at[pl.ds(TB, RB)], out_ref.at[sl])
            pltpu.sync_copy(rows.at[pl.ds(TA, RB)], acc.at[sl])
            return carry
        lax.fori_loop(0, NRB, dblk, 0)

    def layer(_, carry):
        scatter_phase(xuc, ep_ur)
        plsc.subcore_barrier()
        drain(routc, xrc)
        plsc.subcore_barrier()
        scatter_phase(xrc, ep_ru)
        plsc.subcore_barrier()
        drain(uoutc, xuc)
        plsc.subcore_barrier()
        return carry
    lax.fori_loop(0, 3, layer, 0)


@functools.cache
def _sc_kernel():
    return functools.partial(
        pl.kernel,
        out_type=[
            jax.ShapeDtypeStruct((NC, NP, C), jnp.float32),
            jax.ShapeDtypeStruct((NC, NP, C), jnp.float32),
            jax.ShapeDtypeStruct((NC, NP, C), jnp.float32),
            jax.ShapeDtypeStruct((NC, NP, C), jnp.float32),
        ],
        mesh=plsc.VectorSubcoreMesh(core_axis_name="c", subcore_axis_name="s",
                                    num_cores=NC, num_subcores=NS),
        compiler_params=pltpu.CompilerParams(use_tc_tiling_on_sc=False),
        scratch_types=[
            pltpu.VMEM_SHARED((NP, C), jnp.float32),
            pltpu.VMEM((SLOTS * EC, C), jnp.float32),
            pltpu.VMEM((SLOTS, 3, EC), jnp.int32),
            pltpu.SemaphoreType.DMA((SLOTS,)),
            pltpu.SemaphoreType.DMA((SLOTS,)),
            pltpu.SemaphoreType.DMA((SLOTS,)),
        ],
    )(_body)


def _prep_edges(edges, w):
    pad_idx = (jnp.arange(EPAD, dtype=jnp.int32) * 97) % N
    se = jnp.concatenate([edges[0], pad_idx]).reshape(EROWS2, EC)
    de = jnp.concatenate([edges[1], pad_idx]).reshape(EROWS2, EC)
    wi = jax.lax.bitcast_convert_type(
        jnp.concatenate([w, jnp.zeros((EPAD,), jnp.float32)]), jnp.int32)
    return jnp.stack([se, de, wi.reshape(EROWS2, EC)], axis=1)


def kernel(recipe_x, usr_rcp_edges, rcp_usr_edges, usr_rcp_weights,
           rcp_usr_weights, usr_emb, rcp_emb):
    rcp_x0 = jnp.concatenate([rcp_emb, recipe_x], axis=1)
    zpad = jnp.zeros((NP - N, D), jnp.float32)
    usr_p = jnp.concatenate([usr_emb, zpad], axis=0)
    rcp_p = jnp.concatenate([rcp_x0, zpad], axis=0)
    ux0 = jnp.stack([usr_p[:, :C], usr_p[:, C:]])
    rx0 = jnp.stack([rcp_p[:, :C], rcp_p[:, C:]])
    ep_ur = _prep_edges(usr_rcp_edges, usr_rcp_weights)
    ep_ru = _prep_edges(rcp_usr_edges, rcp_usr_weights)
    uout, rout, _, _ = _sc_kernel()(ux0, rx0, ep_ur, ep_ru)
    usr_out = jnp.concatenate([uout[0, :N], uout[1, :N]], axis=1)
    rec_out = jnp.concatenate([rout[0, :N], rout[1, :N]], axis=1)
    return (usr_out, rec_out)


# speedup vs baseline: 1.6099x; 1.6099x over previous
import functools

import jax
import jax.numpy as jnp
from jax import lax
from jax.experimental import pallas as pl
from jax.experimental.pallas import tpu as pltpu
from jax.experimental.pallas import tpu_sc as plsc

N = 50000
NP = 50048
D = 64
C = 32
E = 800000
NC, NS, L = 2, 16, 16

EPAD = 5568 * 144 - E
EC = 144
EROWS2 = 5568
UPT = EROWS2 // NS
SLOTS = 6
ROUNDS = UPT // SLOTS
RPT_N = NP // NS
RB = 184
NRB = RPT_N // RB
ALPHA = 1.0 / 4.0


def _bcast(w16, e):
    return jnp.take_along_axis(w16, jnp.full((16,), e, jnp.int32), axis=0)


def _body(ux0, rx0, ep_ur, ep_ru,
          uout, rout, xu, xr,
          acc, rows, epk, gsem, ssem, isem):
    cid = lax.axis_index("c")
    sid = lax.axis_index("s")
    rbase = sid * RPT_N
    TA, TB = 0, RB

    ux0c, rx0c = ux0.at[cid], rx0.at[cid]
    uoutc, routc = uout.at[cid], rout.at[cid]
    xuc, xrc = xu.at[cid], xr.at[cid]

    def initblk(b, carry):
        off = pl.multiple_of(rbase + b * RB, 8)
        sl = pl.ds(off, RB)
        pltpu.sync_copy(ux0c.at[sl], rows.at[pl.ds(TA, RB)])
        pltpu.sync_copy(rx0c.at[sl], rows.at[pl.ds(TB, RB)])
        pltpu.sync_copy(rows.at[pl.ds(TA, RB)], xuc.at[sl])

        def irow(r, c2):
            rows[TA + r, 0:16] = rows[TA + r, 0:16] * ALPHA
            rows[TA + r, 16:32] = rows[TA + r, 16:32] * ALPHA
            rows[TB + r, 0:16] = rows[TB + r, 0:16] * ALPHA
            rows[TB + r, 16:32] = rows[TB + r, 16:32] * ALPHA
            return c2
        lax.fori_loop(0, RB, irow, 0, unroll=2)
        pltpu.sync_copy(rows.at[pl.ds(TA, RB)], uoutc.at[sl])
        pltpu.sync_copy(rows.at[pl.ds(TB, RB)], routc.at[sl])
        return carry
    lax.fori_loop(0, NRB, initblk, 0)

    def zrow(r, carry):
        z16 = lax.broadcast(jnp.float32(0.0), (16,))
        rows[r, 0:16] = z16
        rows[r, 16:32] = z16
        return carry
    lax.fori_loop(0, RB, zrow, 0, unroll=8)

    def zacc(b, carry):
        off = pl.multiple_of(rbase + b * RB, 8)
        pltpu.sync_copy(rows.at[pl.ds(TA, RB)], acc.at[pl.ds(off, RB)])
        return carry
    lax.fori_loop(0, NRB, zacc, 0)
    plsc.subcore_barrier()

    ubase = sid * UPT

    def scatter_phase(xsrc, ep):
        def load_idx(u, s):
            pltpu.async_copy(ep.at[pl.ds(u, 1)], epk.at[pl.ds(s, 1)],
                             isem.at[s])

        def wait_idx(s):
            pltpu.make_async_copy(ep.at[pl.ds(ubase, 1)],
                                  epk.at[pl.ds(s, 1)], isem.at[s]).wait()

        def gather(s):
            pltpu.async_copy(xsrc.at[epk.at[s, 0]],
                             rows.at[pl.ds(s * EC, EC)], gsem.at[s])

        def wait_gather(s):
            pltpu.make_async_copy(xsrc.at[epk.at[s, 0]],
                                  rows.at[pl.ds(s * EC, EC)],
                                  gsem.at[s]).wait()

        def scatter(s):
            pltpu.async_copy(rows.at[pl.ds(s * EC, EC)],
                             acc.at[epk.at[s, 1]], ssem.at[s], add=True)

        def wait_scatter(s):
            pltpu.make_async_copy(rows.at[pl.ds(s * EC, EC)],
                                  acc.at[epk.at[s, 1]], ssem.at[s]).wait()

        def mul_unit(s):
            def mgrp(g, carry2):
                w16 = lax.bitcast_convert_type(epk[s, 2, pl.ds(g * 16, 16)],
                                               jnp.float32)
                base = s * EC + g * 16
                for e in range(16):
                    wb = _bcast(w16, e)
                    rows[base + e, 0:16] = rows[base + e, 0:16] * wb
                    rows[base + e, 16:32] = rows[base + e, 16:32] * wb
                return carry2
            lax.fori_loop(0, EC // 16, mgrp, 0)

        def zslot(r, carry):
            z16 = lax.broadcast(jnp.float32(0.0), (16,))
            rows[4 * EC + r, 0:16] = z16
            rows[4 * EC + r, 16:32] = z16
            return carry
        lax.fori_loop(0, 2 * EC, zslot, 0, unroll=4)
        for s in range(3):
            pltpu.sync_copy(ep.at[pl.ds(ubase + s, 1)], epk.at[pl.ds(s, 1)])
            gather(s)
        load_idx(ubase + 3, 3)
        for s in (4, 5):
            pltpu.sync_copy(ep.at[pl.ds(ubase, 1)], epk.at[pl.ds(s, 1)])
            scatter(s)

        def round_body(r, carry):
            h0 = r * SLOTS
            for k in range(SLOTS):
                k3 = (k + 3) % SLOTS
                k4 = (k + 4) % SLOTS
                wait_gather(k)
                mul_unit(k)
                scatter(k)
                wait_idx(k3)
                gather(k3)
                wait_scatter(k4)
                load_idx(ubase + lax.rem(h0 + k + 4, UPT), k4)
            return carry
        lax.fori_loop(0, ROUNDS, round_body, 0)

        for s in range(3):
            wait_gather(s)
        wait_idx(3)
        wait_scatter(4)
        wait_scatter(5)

    def drain(out_ref, xdst):
        def dblk(b, carry):
            off = pl.multiple_of(rbase + b * RB, 8)
            sl = pl.ds(off, RB)
            pltpu.sync_copy(acc.at[sl], rows.at[pl.ds(TA, RB)])
            pltpu.sync_copy(rows.at[pl.ds(TA, RB)], xdst.at[sl])
            pltpu.sync_copy(out_ref.at[sl], rows.at[pl.ds(TB, RB)])

            def urow(r, carry2):
                z16 = lax.broadcast(jnp.float32(0.0), (16,))
                a0 = rows[TA + r, 0:16]
                a1 = rows[TA + r, 16:32]
                rows[TB + r, 0:16] = rows[TB + r, 0:16] + a0 * ALPHA
                rows[TB + r, 16:32] = rows[TB + r, 16:32] + a1 * ALPHA
                rows[TA + r, 0:16] = z16
                rows[TA + r, 16:32] = z16
                return carry2
            lax.fori_loop(0, RB, urow, 0, unroll=2)
            pltpu.sync_copy(rows.at[pl.ds(TB, RB)], out_ref.at[sl])
            pltpu.sync_copy(rows.at[pl.ds(TA, RB)], acc.at[sl])
            return carry
        lax.fori_loop(0, NRB, dblk, 0)

    def layer(_, carry):
        scatter_phase(xuc, ep_ur)
        plsc.subcore_barrier()
        drain(routc, xrc)
        plsc.subcore_barrier()
        scatter_phase(xrc, ep_ru)
        plsc.subcore_barrier()
        drain(uoutc, xuc)
        plsc.subcore_barrier()
        return carry
    lax.fori_loop(0, 3, layer, 0)


@functools.cache
def _sc_kernel():
    return functools.partial(
        pl.kernel,
        out_type=[
            jax.ShapeDtypeStruct((NC, NP, C), jnp.float32),
            jax.ShapeDtypeStruct((NC, NP, C), jnp.float32),
            jax.ShapeDtypeStruct((NC, NP, C), jnp.float32),
            jax.ShapeDtypeStruct((NC, NP, C), jnp.float32),
        ],
        mesh=plsc.VectorSubcoreMesh(core_axis_name="c", subcore_axis_name="s",
                                    num_cores=NC, num_subcores=NS),
        compiler_params=pltpu.CompilerParams(use_tc_tiling_on_sc=False),
        scratch_types=[
            pltpu.VMEM_SHARED((NP, C), jnp.float32),
            pltpu.VMEM((SLOTS * EC, C), jnp.float32),
            pltpu.VMEM((SLOTS, 3, EC), jnp.int32),
            pltpu.SemaphoreType.DMA((SLOTS,)),
            pltpu.SemaphoreType.DMA((SLOTS,)),
            pltpu.SemaphoreType.DMA((SLOTS,)),
        ],
    )(_body)


def _prep_edges(edges, w):
    pad_idx = (jnp.arange(EPAD, dtype=jnp.int32) * 97) % N
    se = jnp.concatenate([edges[0], pad_idx]).reshape(EROWS2, EC)
    de = jnp.concatenate([edges[1], pad_idx]).reshape(EROWS2, EC)
    wi = jax.lax.bitcast_convert_type(
        jnp.concatenate([w, jnp.zeros((EPAD,), jnp.float32)]), jnp.int32)
    return jnp.stack([se, de, wi.reshape(EROWS2, EC)], axis=1)


def kernel(recipe_x, usr_rcp_edges, rcp_usr_edges, usr_rcp_weights,
           rcp_usr_weights, usr_emb, rcp_emb):
    rcp_x0 = jnp.concatenate([rcp_emb, recipe_x], axis=1)
    zpad = jnp.zeros((NP - N, D), jnp.float32)
    usr_p = jnp.concatenate([usr_emb, zpad], axis=0)
    rcp_p = jnp.concatenate([rcp_x0, zpad], axis=0)
    ux0 = jnp.stack([usr_p[:, :C], usr_p[:, C:]])
    rx0 = jnp.stack([rcp_p[:, :C], rcp_p[:, C:]])
    ep_ur = _prep_edges(usr_rcp_edges, usr_rcp_weights)
    ep_ru = _prep_edges(rcp_usr_edges, rcp_usr_weights)
    uout, rout, _, _ = _sc_kernel()(ux0, rx0, ep_ur, ep_ru)
    usr_out = jnp.concatenate([uout[0, :N], uout[1, :N]], axis=1)
    rec_out = jnp.concatenate([rout[0, :N], rout[1, :N]], axis=1)
    return (usr_out, rec_out)
